# Initial kernel scaffold; baseline (speedup 1.0000x reference)
import jax
import jax.numpy as jnp
from jax.experimental import pallas as pl


def kernel(user_id, user_age, user_gender, recall_item_id, recall_cate_id, hist_item_id, hist_cate_id, ctx_hour, ctx_device, history_mask, params):
    raise NotImplementedError("write your pallas kernel here")


# speedup vs baseline: 1.0760x; 1.0760x over previous
import functools

import jax
import jax.numpy as jnp
from jax import lax
from jax.experimental import pallas as pl
from jax.experimental.pallas import tpu as pltpu
from jax.experimental.pallas import tpu_sc as plsc

_NC = 2
_NS = 16
_NW = _NC * _NS
_CHUNK = 128


def _sc_gather(emb_item, emb_cate, emb_user, emb_age, emb_gender, emb_hour,
               emb_device, hist_item_idx, hist_cate_idx, small_idx, B, T, D):
    BT = B * T
    rw = BT // _NW
    nch = rw // _CHUNK
    nchp = ((nch + 7) // 8) * 8
    sb = B // _NW
    hidx3 = hist_item_idx.reshape(_NW, nch, _CHUNK)
    cidx3 = hist_cate_idx.reshape(_NW, nch, _CHUNK)
    pad = ((0, 0), (0, nchp - nch), (0, 0))
    hidx3 = jnp.pad(hidx3, pad)
    cidx3 = jnp.pad(cidx3, pad)
    small_idx = jnp.pad(small_idx, ((0, 0), (0, 1), (0, 0)))

    mesh = plsc.VectorSubcoreMesh(core_axis_name="c", subcore_axis_name="s")
    big = jax.ShapeDtypeStruct((BT, D), jnp.float32)
    sml = jax.ShapeDtypeStruct((B, D), jnp.float32)

    @functools.partial(
        pl.kernel,
        mesh=mesh,
        compiler_params=pltpu.CompilerParams(use_tc_tiling_on_sc=False),
        out_type=[big, big] + [sml] * 7,
        scratch_types=[
            pltpu.VMEM((nchp, _CHUNK), jnp.int32),
            pltpu.VMEM((nchp, _CHUNK), jnp.int32),
            pltpu.VMEM((8, _CHUNK), jnp.int32),
            pltpu.VMEM((_CHUNK, D), jnp.float32),
            pltpu.VMEM((_CHUNK, D), jnp.float32),
            pltpu.SemaphoreType.DMA,
            pltpu.SemaphoreType.DMA,
        ],
    )
    def gather_kernel(item_hbm, cate_hbm, user_hbm, age_hbm, gender_hbm,
                      hour_hbm, device_hbm, hidx_hbm, cidx_hbm, sidx_hbm,
                      hi_out, hc_out, u_out, a_out, g_out, ri_out, rc_out,
                      ho_out, de_out,
                      hidx_v, cidx_v, sidx_v, rows_a, rows_b, sem_a, sem_b):
        wid = lax.axis_index("s") * _NC + lax.axis_index("c")
        hbase = wid * rw
        sbase = wid * sb
        pltpu.sync_copy(hidx_hbm.at[wid], hidx_v)
        pltpu.sync_copy(cidx_hbm.at[wid], cidx_v)
        pltpu.sync_copy(sidx_hbm.at[wid], sidx_v)

        def chunk(j, carry):
            base = hbase + j * _CHUNK
            cpa = pltpu.async_copy(item_hbm.at[hidx_v.at[j]], rows_a, sem_a)
            cpb = pltpu.async_copy(cate_hbm.at[cidx_v.at[j]], rows_b, sem_b)
            cpa.wait()
            pltpu.sync_copy(rows_a, hi_out.at[pl.ds(base, _CHUNK)])
            cpb.wait()
            pltpu.sync_copy(rows_b, hc_out.at[pl.ds(base, _CHUNK)])
            return carry

        lax.fori_loop(0, nch, chunk, 0)

        def small(g, table, out_ref):
            pltpu.async_copy(table.at[sidx_v.at[g]], rows_a, sem_a).wait()
            pltpu.sync_copy(rows_a, out_ref.at[pl.ds(sbase, _CHUNK)])

        small(0, user_hbm, u_out)
        small(1, age_hbm, a_out)
        small(2, gender_hbm, g_out)
        small(3, item_hbm, ri_out)
        small(4, cate_hbm, rc_out)
        small(5, hour_hbm, ho_out)
        small(6, device_hbm, de_out)

    return gather_kernel(emb_item, emb_cate, emb_user, emb_age, emb_gender,
                         emb_hour, emb_device, hidx3, cidx3, small_idx)


def _z1(hi, hc, qv, w1, b1):
    h2 = jnp.concatenate([hi, hc], axis=-1)
    d2 = h2.shape[-1]
    wh = w1[0:d2] - w1[2 * d2:3 * d2]
    wq = w1[d2:2 * d2] + w1[2 * d2:3 * d2]
    wd = w1[3 * d2:4 * d2]
    whd = jnp.concatenate([wh, wd], axis=0)
    qh = h2 * qv[:, None, :]
    hqh = jnp.concatenate([h2, qh], axis=-1)
    bb, tt = h2.shape[0], h2.shape[1]
    z = jnp.dot(hqh.reshape(bb * tt, 2 * d2), whd,
                preferred_element_type=jnp.float32)
    z = z.reshape(bb, tt, z.shape[-1])
    z = z + jnp.dot(qv, wq, preferred_element_type=jnp.float32)[:, None, :]
    return z + b1.reshape(1, 1, -1), h2


def _dice_from_stats(z, s1, s2, n):
    mean = s1 * (1.0 / n)
    var = (s2 - n * mean * mean) * (1.0 / (n - 1))
    std = jnp.sqrt(jnp.maximum(var, 0.0))
    xn = (z - mean) / (std + 1e-8)
    p = jax.nn.sigmoid(xn)
    return z * (0.01 + 0.99 * p)


def _tc1_body(hi_ref, hc_ref, ri_ref, rc_ref, w1_ref, b1_ref, stats_ref,
              acc_ref):
    i = pl.program_id(0)
    qv = jnp.concatenate([ri_ref[...], rc_ref[...]], axis=-1)
    z, _ = _z1(hi_ref[...], hc_ref[...], qv, w1_ref[...], b1_ref[...])

    @pl.when(i == 0)
    def _():
        acc_ref[...] = jnp.zeros_like(acc_ref)

    acc_ref[0, :, :] += jnp.sum(z, axis=0)
    acc_ref[1, :, :] += jnp.sum(z * z, axis=0)

    @pl.when(i == pl.num_programs(0) - 1)
    def _():
        stats_ref[...] = acc_ref[...]


def _tc2_body(hi_ref, hc_ref, ri_ref, rc_ref, u_ref, a_ref, g_ref, ho_ref,
              de_ref, mask_ref, stats1_ref,
              w1_ref, b1_ref, w2t_ref, b2_ref, mw1_ref, mb1_ref,
              z2_ref, stats2_ref, acc_ref, *, btot):
    i = pl.program_id(0)
    qv = jnp.concatenate([ri_ref[...], rc_ref[...]], axis=-1)
    z, h2 = _z1(hi_ref[...], hc_ref[...], qv, w1_ref[...], b1_ref[...])
    stats1 = stats1_ref[...]
    act = _dice_from_stats(z, stats1[0][None], stats1[1][None], btot)
    scores = jnp.sum(act * w2t_ref[...].reshape(1, 1, -1), axis=-1)
    scores = (scores + b2_ref[0, 0]) * mask_ref[...]
    weighted = jnp.sum(scores[:, :, None] * h2, axis=1)
    x = jnp.concatenate([u_ref[...], a_ref[...], g_ref[...], ho_ref[...],
                         de_ref[...], qv, weighted], axis=-1)
    z2 = jnp.dot(x, mw1_ref[...], preferred_element_type=jnp.float32)
    z2 = z2 + mb1_ref[...]
    z2_ref[...] = z2

    @pl.when(i == 0)
    def _():
        acc_ref[...] = jnp.zeros_like(acc_ref)

    acc_ref[0, :] += jnp.sum(z2, axis=0)
    acc_ref[1, :] += jnp.sum(z2 * z2, axis=0)

    @pl.when(i == pl.num_programs(0) - 1)
    def _():
        stats2_ref[...] = acc_ref[...]


def _tc3_body(z2_ref, stats2_ref, mw2_ref, mb2_ref, mw3_ref, mb3_ref,
              mw4_ref, mb4_ref, out_ref, *, btot):
    z2 = z2_ref[...]
    stats2 = stats2_ref[...]
    x = _dice_from_stats(z2, stats2[0][None], stats2[1][None], btot)

    def dice_full(z):
        s1 = jnp.sum(z, axis=0, keepdims=True)
        s2 = jnp.sum(z * z, axis=0, keepdims=True)
        return _dice_from_stats(z, s1, s2, btot)

    z3 = jnp.dot(x, mw2_ref[...], preferred_element_type=jnp.float32)
    x3 = dice_full(z3 + mb2_ref[...])
    z4 = jnp.dot(x3, mw3_ref[...], preferred_element_type=jnp.float32)
    x4 = dice_full(z4 + mb3_ref[...])
    logits = (x4[:, 0:1] * mw4_ref[0, 0] + x4[:, 1:2] * mw4_ref[1, 0]
              + mb4_ref[0, 0])
    out_ref[...] = jax.nn.sigmoid(logits)


def kernel(user_id, user_age, user_gender, recall_item_id, recall_cate_id,
           hist_item_id, hist_cate_id, ctx_hour, ctx_device, history_mask,
           params):
    p = params
    B, T = hist_item_id.shape
    D = p['emb_item_id'].shape[1]
    i32 = jnp.int32

    small_idx = jnp.stack([
        user_id.astype(i32), user_age.astype(i32), user_gender.astype(i32),
        recall_item_id.astype(i32), recall_cate_id.astype(i32),
        ctx_hour.astype(i32), ctx_device.astype(i32)], axis=0)
    small_idx = small_idx.reshape(7, _NW, B // _NW).transpose(1, 0, 2)

    hi, hc, ue, ae, ge, ri, rc, ho, de = _sc_gather(
        p['emb_item_id'], p['emb_cate_id'], p['emb_user_id'],
        p['emb_user_age'], p['emb_user_gender'], p['emb_hour'],
        p['emb_device'], hist_item_id.astype(i32).reshape(-1),
        hist_cate_id.astype(i32).reshape(-1), small_idx, B, T, D)

    return _dense_forward(hi.reshape(B, T, D), hc.reshape(B, T, D),
                          ri, rc, ue, ae, ge, ho, de, history_mask, p)


def _dense_forward(hi3, hc3, ri, rc, ue, ae, ge, ho, de, history_mask, p):
    B, T, D = hi3.shape
    NH = p['att_w1'].shape[1]
    BB = 256
    NB = B // BB

    b1 = p['att_b1'].reshape(1, NH)
    w2t = p['att_w2'].reshape(1, NH)
    b2 = p['att_b2'].reshape(1, 1)
    mb1 = p['mlp_b1'].reshape(1, -1)
    mb2 = p['mlp_b2'].reshape(1, -1)
    mb3 = p['mlp_b3'].reshape(1, -1)
    mb4 = p['mlp_b4'].reshape(1, 1)
    M1 = p['mlp_w1'].shape[1]
    M2 = p['mlp_w2'].shape[1]

    hblk = pl.BlockSpec((BB, T, D), lambda i: (i, 0, 0))
    bblk = pl.BlockSpec((BB, D), lambda i: (i, 0))

    stats1 = pl.pallas_call(
        _tc1_body,
        grid=(NB,),
        in_specs=[
            hblk, hblk, bblk, bblk,
            pl.BlockSpec((8 * D, NH), lambda i: (0, 0)),
            pl.BlockSpec((1, NH), lambda i: (0, 0)),
        ],
        out_specs=pl.BlockSpec((2, T, NH), lambda i: (0, 0, 0)),
        out_shape=jax.ShapeDtypeStruct((2, T, NH), jnp.float32),
        scratch_shapes=[pltpu.VMEM((2, T, NH), jnp.float32)],
    )(hi3, hc3, ri, rc, p['att_w1'], b1)

    z2, stats2 = pl.pallas_call(
        functools.partial(_tc2_body, btot=B),
        grid=(NB,),
        in_specs=[
            hblk, hblk, bblk, bblk, bblk, bblk, bblk, bblk, bblk,
            pl.BlockSpec((BB, T), lambda i: (i, 0)),
            pl.BlockSpec((2, T, NH), lambda i: (0, 0, 0)),
            pl.BlockSpec((8 * D, NH), lambda i: (0, 0)),
            pl.BlockSpec((1, NH), lambda i: (0, 0)),
            pl.BlockSpec((1, NH), lambda i: (0, 0)),
            pl.BlockSpec((1, 1), lambda i: (0, 0)),
            pl.BlockSpec((9 * D, M1), lambda i: (0, 0)),
            pl.BlockSpec((1, M1), lambda i: (0, 0)),
        ],
        out_specs=[
            pl.BlockSpec((BB, M1), lambda i: (i, 0)),
            pl.BlockSpec((2, M1), lambda i: (0, 0)),
        ],
        out_shape=[
            jax.ShapeDtypeStruct((B, M1), jnp.float32),
            jax.ShapeDtypeStruct((2, M1), jnp.float32),
        ],
        scratch_shapes=[pltpu.VMEM((2, M1), jnp.float32)],
    )(hi3, hc3, ri, rc, ue, ae, ge, ho, de, history_mask, stats1,
      p['att_w1'], b1, w2t, b2, p['mlp_w1'], mb1)

    out = pl.pallas_call(
        functools.partial(_tc3_body, btot=B),
        grid=(1,),
        in_specs=[
            pl.BlockSpec((B, M1), lambda i: (0, 0)),
            pl.BlockSpec((2, M1), lambda i: (0, 0)),
            pl.BlockSpec((M1, M2), lambda i: (0, 0)),
            pl.BlockSpec((1, M2), lambda i: (0, 0)),
            pl.BlockSpec((M2, 2), lambda i: (0, 0)),
            pl.BlockSpec((1, 2), lambda i: (0, 0)),
            pl.BlockSpec((2, 1), lambda i: (0, 0)),
            pl.BlockSpec((1, 1), lambda i: (0, 0)),
        ],
        out_specs=pl.BlockSpec((B, 1), lambda i: (0, 0)),
        out_shape=jax.ShapeDtypeStruct((B, 1), jnp.float32),
    )(z2, stats2, p['mlp_w2'], mb2, p['mlp_w3'], mb3, p['mlp_w4'], mb4)

    return out[:, 0]


# speedup vs baseline: 1.4324x; 1.3312x over previous
import functools

import jax
import jax.numpy as jnp
from jax import lax
from jax.experimental import pallas as pl
from jax.experimental.pallas import tpu as pltpu
from jax.experimental.pallas import tpu_sc as plsc

_NC = 2
_NS = 16
_NW = _NC * _NS
_CHUNK = 128


def _sc_gather(emb_item, emb_cate, emb_user, emb_age, emb_gender, emb_hour,
               emb_device, hist_item_idx, hist_cate_idx, small_idx, B, T, D):
    BT = B * T
    rw = BT // _NW
    nch = rw // _CHUNK
    nchp = ((nch + 7) // 8) * 8
    sb = B // _NW
    hidx3 = hist_item_idx.reshape(_NW, nch, _CHUNK)
    cidx3 = hist_cate_idx.reshape(_NW, nch, _CHUNK)
    pad = ((0, 0), (0, nchp - nch), (0, 0))
    hidx3 = jnp.pad(hidx3, pad)
    cidx3 = jnp.pad(cidx3, pad)
    small_idx = jnp.pad(small_idx, ((0, 0), (0, 1), (0, 0)))

    mesh = plsc.VectorSubcoreMesh(core_axis_name="c", subcore_axis_name="s")
    big = jax.ShapeDtypeStruct((BT, D), jnp.float32)
    sml = jax.ShapeDtypeStruct((B, D), jnp.float32)

    @functools.partial(
        pl.kernel,
        mesh=mesh,
        compiler_params=pltpu.CompilerParams(use_tc_tiling_on_sc=False),
        out_type=[big, big] + [sml] * 7,
        scratch_types=[
            pltpu.VMEM((nchp, _CHUNK), jnp.int32),
            pltpu.VMEM((nchp, _CHUNK), jnp.int32),
            pltpu.VMEM((8, _CHUNK), jnp.int32),
            pltpu.VMEM((_CHUNK, D), jnp.float32),
            pltpu.VMEM((_CHUNK, D), jnp.float32),
            pltpu.SemaphoreType.DMA,
            pltpu.SemaphoreType.DMA,
        ],
    )
    def gather_kernel(item_hbm, cate_hbm, user_hbm, age_hbm, gender_hbm,
                      hour_hbm, device_hbm, hidx_hbm, cidx_hbm, sidx_hbm,
                      hi_out, hc_out, u_out, a_out, g_out, ri_out, rc_out,
                      ho_out, de_out,
                      hidx_v, cidx_v, sidx_v, rows_a, rows_b, sem_a, sem_b):
        wid = lax.axis_index("s") * _NC + lax.axis_index("c")
        hbase = wid * rw
        sbase = wid * sb
        pltpu.sync_copy(hidx_hbm.at[wid], hidx_v)
        pltpu.sync_copy(cidx_hbm.at[wid], cidx_v)
        pltpu.sync_copy(sidx_hbm.at[wid], sidx_v)

        def chunk(j, carry):
            base = hbase + j * _CHUNK
            cpa = pltpu.async_copy(item_hbm.at[hidx_v.at[j]], rows_a, sem_a)
            cpb = pltpu.async_copy(cate_hbm.at[cidx_v.at[j]], rows_b, sem_b)
            cpa.wait()
            pltpu.sync_copy(rows_a, hi_out.at[pl.ds(base, _CHUNK)])
            cpb.wait()
            pltpu.sync_copy(rows_b, hc_out.at[pl.ds(base, _CHUNK)])
            return carry

        lax.fori_loop(0, nch, chunk, 0)

        def small(g, table, out_ref):
            pltpu.async_copy(table.at[sidx_v.at[g]], rows_a, sem_a).wait()
            pltpu.sync_copy(rows_a, out_ref.at[pl.ds(sbase, _CHUNK)])

        small(0, user_hbm, u_out)
        small(1, age_hbm, a_out)
        small(2, gender_hbm, g_out)
        small(3, item_hbm, ri_out)
        small(4, cate_hbm, rc_out)
        small(5, hour_hbm, ho_out)
        small(6, device_hbm, de_out)

    return gather_kernel(emb_item, emb_cate, emb_user, emb_age, emb_gender,
                         emb_hour, emb_device, hidx3, cidx3, small_idx)


def _dice_from_stats(z, s1, s2, n):
    mean = s1 * (1.0 / n)
    var = (s2 - n * mean * mean) * (1.0 / (n - 1))
    std = jnp.sqrt(jnp.maximum(var, 0.0))
    xn = (z - mean) / (std + 1e-8)
    p = jax.nn.sigmoid(xn)
    return z * (0.01 + 0.99 * p)


def _tca_body(hi_ref, hc_ref, ri_ref, rc_ref, u_ref, a_ref, g_ref, ho_ref,
              de_ref, mask_ref, w1_ref, b1_ref, w2t_ref, b2_ref, mw1_ref,
              mb1_ref, z2_ref, stats2_ref, wacc_ref, *, btot):
    t = pl.program_id(0)
    hi = hi_ref[...]
    hc = hc_ref[...]
    h2 = jnp.concatenate([hi, hc], axis=-1)
    qv = jnp.concatenate([ri_ref[...], rc_ref[...]], axis=-1)
    w1 = w1_ref[...]
    d2 = h2.shape[-1]
    wh = w1[0:d2] - w1[2 * d2:3 * d2]
    wq = w1[d2:2 * d2] + w1[2 * d2:3 * d2]
    wd = w1[3 * d2:4 * d2]
    whd = jnp.concatenate([wh, wd], axis=0)
    hqh = jnp.concatenate([h2, h2 * qv], axis=-1)
    z = jnp.dot(hqh, whd, preferred_element_type=jnp.float32)
    z = z + jnp.dot(qv, wq, preferred_element_type=jnp.float32)
    z = z + b1_ref[...]
    s1 = jnp.sum(z, axis=0, keepdims=True)
    s2 = jnp.sum(z * z, axis=0, keepdims=True)
    act = _dice_from_stats(z, s1, s2, btot)
    scores = jnp.sum(act * w2t_ref[...], axis=1, keepdims=True)
    scores = (scores + b2_ref[0, 0]) * mask_ref[...]

    @pl.when(t == 0)
    def _():
        wacc_ref[...] = jnp.zeros_like(wacc_ref)

    wacc_ref[...] += scores * h2

    @pl.when(t == pl.num_programs(0) - 1)
    def _():
        x = jnp.concatenate([u_ref[...], a_ref[...], g_ref[...], ho_ref[...],
                             de_ref[...], qv, wacc_ref[...]], axis=-1)
        z2 = jnp.dot(x, mw1_ref[...], preferred_element_type=jnp.float32)
        z2 = z2 + mb1_ref[...]
        z2_ref[...] = z2
        stats2_ref[0:1, :] = jnp.sum(z2, axis=0, keepdims=True)
        stats2_ref[1:2, :] = jnp.sum(z2 * z2, axis=0, keepdims=True)


def _tc3_body(z2_ref, stats2_ref, mw2_ref, mb2_ref, mw3_ref, mb3_ref,
              mw4_ref, mb4_ref, out_ref, *, btot):
    z2 = z2_ref[...]
    stats2 = stats2_ref[...]
    x = _dice_from_stats(z2, stats2[0][None], stats2[1][None], btot)

    def dice_full(z):
        s1 = jnp.sum(z, axis=0, keepdims=True)
        s2 = jnp.sum(z * z, axis=0, keepdims=True)
        return _dice_from_stats(z, s1, s2, btot)

    z3 = jnp.dot(x, mw2_ref[...], preferred_element_type=jnp.float32)
    x3 = dice_full(z3 + mb2_ref[...])
    z4 = jnp.dot(x3, mw3_ref[...], preferred_element_type=jnp.float32)
    x4 = dice_full(z4 + mb3_ref[...])
    logits = (x4[:, 0:1] * mw4_ref[0, 0] + x4[:, 1:2] * mw4_ref[1, 0]
              + mb4_ref[0, 0])
    out_ref[...] = jax.nn.sigmoid(logits)


def kernel(user_id, user_age, user_gender, recall_item_id, recall_cate_id,
           hist_item_id, hist_cate_id, ctx_hour, ctx_device, history_mask,
           params):
    p = params
    B, T = hist_item_id.shape
    D = p['emb_item_id'].shape[1]
    i32 = jnp.int32

    small_idx = jnp.stack([
        user_id.astype(i32), user_age.astype(i32), user_gender.astype(i32),
        recall_item_id.astype(i32), recall_cate_id.astype(i32),
        ctx_hour.astype(i32), ctx_device.astype(i32)], axis=0)
    small_idx = small_idx.reshape(7, _NW, B // _NW).transpose(1, 0, 2)

    hi, hc, ue, ae, ge, ri, rc, ho, de = _sc_gather(
        p['emb_item_id'], p['emb_cate_id'], p['emb_user_id'],
        p['emb_user_age'], p['emb_user_gender'], p['emb_hour'],
        p['emb_device'],
        hist_item_id.astype(i32).T.reshape(-1),
        hist_cate_id.astype(i32).T.reshape(-1),
        small_idx, B, T, D)

    return _dense_forward(hi, hc, ri, rc, ue, ae, ge, ho, de, history_mask,
                          p, T)


def _dense_forward(hi, hc, ri, rc, ue, ae, ge, ho, de, history_mask, p, T):
    B = ri.shape[0]
    D = ri.shape[1]
    NH = p['att_w1'].shape[1]

    b1 = p['att_b1'].reshape(1, NH)
    w2t = p['att_w2'].reshape(1, NH)
    b2 = p['att_b2'].reshape(1, 1)
    mb1 = p['mlp_b1'].reshape(1, -1)
    mb2 = p['mlp_b2'].reshape(1, -1)
    mb3 = p['mlp_b3'].reshape(1, -1)
    mb4 = p['mlp_b4'].reshape(1, 1)
    M1 = p['mlp_w1'].shape[1]
    M2 = p['mlp_w2'].shape[1]

    hblk = pl.BlockSpec((B, D), lambda t: (t, 0))
    cblk = pl.BlockSpec((B, D), lambda t: (0, 0))

    z2, stats2 = pl.pallas_call(
        functools.partial(_tca_body, btot=B),
        grid=(T,),
        in_specs=[
            hblk, hblk, cblk, cblk, cblk, cblk, cblk, cblk, cblk,
            pl.BlockSpec((B, 1), lambda t: (t, 0)),
            pl.BlockSpec((8 * D, NH), lambda t: (0, 0)),
            pl.BlockSpec((1, NH), lambda t: (0, 0)),
            pl.BlockSpec((1, NH), lambda t: (0, 0)),
            pl.BlockSpec((1, 1), lambda t: (0, 0)),
            pl.BlockSpec((9 * D, M1), lambda t: (0, 0)),
            pl.BlockSpec((1, M1), lambda t: (0, 0)),
        ],
        out_specs=[
            pl.BlockSpec((B, M1), lambda t: (0, 0)),
            pl.BlockSpec((2, M1), lambda t: (0, 0)),
        ],
        out_shape=[
            jax.ShapeDtypeStruct((B, M1), jnp.float32),
            jax.ShapeDtypeStruct((2, M1), jnp.float32),
        ],
        scratch_shapes=[pltpu.VMEM((B, 2 * D), jnp.float32)],
    )(hi, hc, ri, rc, ue, ae, ge, ho, de,
      history_mask.T.reshape(T * B, 1),
      p['att_w1'], b1, w2t, b2, p['mlp_w1'], mb1)

    out = pl.pallas_call(
        functools.partial(_tc3_body, btot=B),
        grid=(1,),
        in_specs=[
            pl.BlockSpec((B, M1), lambda i: (0, 0)),
            pl.BlockSpec((2, M1), lambda i: (0, 0)),
            pl.BlockSpec((M1, M2), lambda i: (0, 0)),
            pl.BlockSpec((1, M2), lambda i: (0, 0)),
            pl.BlockSpec((M2, 2), lambda i: (0, 0)),
            pl.BlockSpec((1, 2), lambda i: (0, 0)),
            pl.BlockSpec((2, 1), lambda i: (0, 0)),
            pl.BlockSpec((1, 1), lambda i: (0, 0)),
        ],
        out_specs=pl.BlockSpec((B, 1), lambda i: (0, 0)),
        out_shape=jax.ShapeDtypeStruct((B, 1), jnp.float32),
    )(z2, stats2, p['mlp_w2'], mb2, p['mlp_w3'], mb3, p['mlp_w4'], mb4)

    return out[:, 0]
